# baseline (speedup 1.0000x reference)
import jax
import jax.numpy as jnp
import numpy as np
from jax.experimental import pallas as pl
from jax.experimental.pallas import tpu as pltpu

_G = 64
_KV = 27
_NP = 50176


def _kernel_maps(coords, n):
    M = _G + 2
    c = coords.astype(jnp.int32) + 1
    keys = c[:, 0] * (M * M) + c[:, 1] * M + c[:, 2]
    order = jnp.argsort(keys)
    skeys = keys[order]
    offs = []
    for dx in (-1, 0, 1):
        for dy in (-1, 0, 1):
            for dz in (-1, 0, 1):
                offs.append(dx * M * M + dy * M + dz)
    offs = jnp.asarray(offs, jnp.int32)
    q = keys[None, :] + offs[:, None]
    pos = jnp.searchsorted(skeys, q)
    posc = jnp.clip(pos, 0, n - 1)
    found = skeys[posc] == q
    nbr = jnp.where(found, order[posc], n)
    return jnp.concatenate(
        [nbr, jnp.full((_KV, _NP - n), n, jnp.int32)], axis=1)


def _mm_acc_body(xg_ref, w_ref, o_ref):
    k = pl.program_id(1)

    @pl.when(k == 0)
    def _():
        o_ref[...] = jnp.zeros_like(o_ref)

    o_ref[...] += jnp.dot(xg_ref[0], w_ref[0],
                          preferred_element_type=jnp.float32)


def _conv_mm(xg, W, tr):
    K, n, cin = xg.shape
    cout = W.shape[2]
    return pl.pallas_call(
        _mm_acc_body,
        grid=(n // tr, K),
        in_specs=[
            pl.BlockSpec((1, tr, cin), lambda i, k: (k, i, 0)),
            pl.BlockSpec((1, cin, cout), lambda i, k: (k, 0, 0)),
        ],
        out_specs=pl.BlockSpec((tr, cout), lambda i, k: (i, 0)),
        out_shape=jax.ShapeDtypeStruct((n, cout), jnp.float32),
        compiler_params=pltpu.CompilerParams(
            dimension_semantics=("parallel", "arbitrary")),
    )(xg, W)


def _fan_mm_body(h_ref, u_ref, o_ref):
    o_ref[0] = jnp.dot(h_ref[...], u_ref[0],
                       preferred_element_type=jnp.float32)


def _fan_mm(h2, U, tr):
    n = h2.shape[0]
    K, cin, cout = U.shape
    return pl.pallas_call(
        _fan_mm_body,
        grid=(n // tr, K),
        in_specs=[
            pl.BlockSpec((tr, cin), lambda i, m: (i, 0)),
            pl.BlockSpec((1, cin, cout), lambda i, m: (m, 0, 0)),
        ],
        out_specs=pl.BlockSpec((1, tr, cout), lambda i, m: (m, i, 0)),
        out_shape=jax.ShapeDtypeStruct((K, n, cout), jnp.float32),
        compiler_params=pltpu.CompilerParams(
            dimension_semantics=("parallel", "arbitrary")),
    )(h2, U)


def kernel(x, coords, W1, W2, W3, W4):
    n = x.shape[0]
    nbr = _kernel_maps(coords, n)

    x_p = jnp.zeros((_NP, 2), x.dtype).at[:n].set(x)
    xg = jnp.stack([x_p[nbr[k]] for k in range(_KV)])
    Xg = xg.transpose(1, 0, 2).reshape(_NP, 2 * _KV)

    Xgg = jnp.stack([Xg[nbr[k]] for k in range(_KV)])

    W1cat = W1.reshape(2 * _KV, 256)
    V2 = jnp.einsum('ac,kcd->kad', W1cat, W2,
                    precision=jax.lax.Precision.HIGHEST)
    h2 = _conv_mm(Xgg, V2, tr=6272)

    W4cat = W4.transpose(1, 0, 2).reshape(256, 2 * _KV)
    U = jnp.einsum('kab,bc->kac', W3, W4cat,
                   precision=jax.lax.Precision.HIGHEST)
    T = _fan_mm(h2, U, tr=6272)

    Tf = T.reshape(_KV * _NP, 2 * _KV)
    y = sum(Tf[nbr[m] + m * _NP] for m in range(_KV))

    Y2 = y.reshape(_NP * _KV, 2)
    out = sum(Y2[nbr[k] * _KV + k] for k in range(_KV))
    return out[:n]
